# baseline (speedup 1.0000x reference)
import functools

import jax
import jax.numpy as jnp
from jax import lax
from jax.experimental import pallas as pl
from jax.experimental.pallas import tpu as pltpu
from jax.experimental.pallas import tpu_sc as plsc

_ENTITY_COUNT = 1000000
_DIM = 64
_MARGIN = 1.0
_BATCH = 16384

_NC = 2
_NS = 16
_NW = _NC * _NS
_BW = _BATCH // _NW
_CH = 128
_NCH = _BW // _CH
_NG = _BW // 16


def _rsqrt16(x):
    i = plsc.bitcast(x, jnp.int32)
    y = plsc.bitcast(jnp.int32(0x5F3759DF) - (i >> 1), jnp.float32)
    for _ in range(3):
        y = y * (1.5 - 0.5 * x * y * y)
    return y


def _body(pos_h, pos_r, pos_t, neg_h, neg_r, neg_t, ent, rel,
          out_loss, out_pd, out_nd,
          idx_h, idx_r, idx_t, hbuf, rbuf, tbuf, pd_buf, nd_buf, loss_buf,
          sem):
    c = lax.axis_index("c")
    s = lax.axis_index("s")
    wid = s * _NC + c
    riota = lax.iota(jnp.int32, 16)
    zeros = jnp.zeros((16,), jnp.float32)

    def compute_set(ih_hbm, ir_hbm, it_hbm, dist_buf):
        base = wid * _NCH
        pltpu.sync_copy(ih_hbm.at[pl.ds(base, _NCH)], idx_h)
        pltpu.sync_copy(ir_hbm.at[pl.ds(base, _NCH)], idx_r)
        pltpu.sync_copy(it_hbm.at[pl.ds(base, _NCH)], idx_t)
        copies = []
        for k in range(_NCH):
            dst = pl.ds(k * _CH, _CH)
            copies.append(pltpu.async_copy(ent.at[idx_h.at[k]], hbuf.at[dst], sem))
            copies.append(pltpu.async_copy(rel.at[idx_r.at[k]], rbuf.at[dst], sem))
            copies.append(pltpu.async_copy(ent.at[idx_t.at[k]], tbuf.at[dst], sem))
        for cp in copies:
            cp.wait()

        def group(g, carry):
            rows = g * 16 + riota

            def normk(k, acc):
                h2, t2 = acc
                kk = jnp.full((16,), k, jnp.int32)
                hv = plsc.load_gather(hbuf, [rows, kk])
                tv = plsc.load_gather(tbuf, [rows, kk])
                return h2 + hv * hv, t2 + tv * tv

            h2, t2 = lax.fori_loop(0, _DIM, normk, (zeros, zeros))
            inv_h = _rsqrt16(h2)
            inv_t = _rsqrt16(t2)

            def distk(k, acc):
                kk = jnp.full((16,), k, jnp.int32)
                hv = plsc.load_gather(hbuf, [rows, kk])
                rv = plsc.load_gather(rbuf, [rows, kk])
                tv = plsc.load_gather(tbuf, [rows, kk])
                d = hv * inv_h + rv - tv * inv_t
                return acc + jnp.abs(d)

            acc = lax.fori_loop(0, _DIM, distk, zeros)
            dist_buf[pl.ds(g * 16, 16)] = acc
            return carry

        lax.fori_loop(0, _NG, group, 0)

    compute_set(pos_h, pos_r, pos_t, pd_buf)
    compute_set(neg_h, neg_r, neg_t, nd_buf)

    def loss_group(g, carry):
        sl = pl.ds(g * 16, 16)
        loss_buf[sl] = jnp.maximum(pd_buf[sl] - nd_buf[sl] + _MARGIN, 0.0)
        return carry

    lax.fori_loop(0, _NG, loss_group, 0)

    obase = pl.ds(wid * _BW, _BW)
    pltpu.sync_copy(loss_buf, out_loss.at[obase])
    pltpu.sync_copy(pd_buf, out_pd.at[obase])
    pltpu.sync_copy(nd_buf, out_nd.at[obase])


_sc_call = pl.kernel(
    _body,
    out_type=(
        jax.ShapeDtypeStruct((_BATCH,), jnp.float32),
        jax.ShapeDtypeStruct((_BATCH,), jnp.float32),
        jax.ShapeDtypeStruct((_BATCH,), jnp.float32),
    ),
    mesh=plsc.VectorSubcoreMesh(
        core_axis_name="c", subcore_axis_name="s",
        num_cores=_NC, num_subcores=_NS),
    scratch_types=[
        pltpu.VMEM((_NCH, _CH), jnp.int32),
        pltpu.VMEM((_NCH, _CH), jnp.int32),
        pltpu.VMEM((_NCH, _CH), jnp.int32),
        pltpu.VMEM((_BW, _DIM), jnp.float32),
        pltpu.VMEM((_BW, _DIM), jnp.float32),
        pltpu.VMEM((_BW, _DIM), jnp.float32),
        pltpu.VMEM((_BW,), jnp.float32),
        pltpu.VMEM((_BW,), jnp.float32),
        pltpu.VMEM((_BW,), jnp.float32),
        pltpu.SemaphoreType.DMA,
    ],
    compiler_params=pltpu.CompilerParams(
        needs_layout_passes=False, use_tc_tiling_on_sc=False),
)


@jax.jit
def kernel(positive_triplets, negative_triplets, entities_emb, relations_emb):
    pos = positive_triplets.astype(jnp.int32)
    neg = negative_triplets.astype(jnp.int32)
    cols = lambda t, j: t[:, j].reshape(_NW * _NCH, _CH)
    return _sc_call(
        cols(pos, 0), cols(pos, 1), cols(pos, 2),
        cols(neg, 0), cols(neg, 1), cols(neg, 2),
        entities_emb, relations_emb)


# speedup vs baseline: 1.2243x; 1.2243x over previous
import functools

import jax
import jax.numpy as jnp
from jax import lax
from jax.experimental import pallas as pl
from jax.experimental.pallas import tpu as pltpu
from jax.experimental.pallas import tpu_sc as plsc

_N_ENT = 1000001
_DIM = 64
_PITCH = 128
_MARGIN = 1.0
_BATCH = 16384

_NC = 2
_NS = 16
_NW = _NC * _NS
_BW = _BATCH // _NW
_SB = 256
_CH = 128
_NCH = _BW // _CH
_NG = _SB // 16

_TBLK = 1024
_TGRID = (_N_ENT + _TBLK - 1) // _TBLK


def _tc_stage_body(ent_ref, rel_ref, o_ref):
    x = ent_ref[...]
    inv = lax.rsqrt(jnp.sum(x * x, axis=0, keepdims=True))
    y = (x * inv).T
    r = rel_ref[...].T
    o_ref[...] = jnp.concatenate([y, r], axis=1)


_tc_stage = pl.pallas_call(
    _tc_stage_body,
    grid=(_TGRID,),
    in_specs=[
        pl.BlockSpec((_DIM, _TBLK), lambda i: (0, i)),
        pl.BlockSpec((_DIM, _TBLK), lambda i: (0, i)),
    ],
    out_specs=pl.BlockSpec((_TBLK, _PITCH), lambda i: (i, 0)),
    out_shape=jax.ShapeDtypeStruct((_N_ENT, _PITCH), jnp.float32),
)


def _sc_body(pos_h, pos_r, pos_t, neg_h, neg_r, neg_t, tab,
             out_loss, out_pd, out_nd,
             idx_h, idx_r, idx_t, hbuf, rbuf, tbuf, pd_buf, nd_buf, loss_buf,
             sem):
    c = lax.axis_index("c")
    s = lax.axis_index("s")
    wid = s * _NC + c
    riota = lax.iota(jnp.int32, 16)
    zeros = jnp.zeros((16,), jnp.float32)

    def compute_set(ih_hbm, ir_hbm, it_hbm, dist_buf):
        base = wid * _NCH
        pltpu.sync_copy(ih_hbm.at[pl.ds(base, _NCH)], idx_h)
        pltpu.sync_copy(ir_hbm.at[pl.ds(base, _NCH)], idx_r)
        pltpu.sync_copy(it_hbm.at[pl.ds(base, _NCH)], idx_t)
        for sub in range(_BW // _SB):
            copies = []
            for k in range(_SB // _CH):
                dst = pl.ds(k * _CH, _CH)
                ic = sub * (_SB // _CH) + k
                copies.append(pltpu.async_copy(tab.at[idx_h.at[ic]], hbuf.at[dst], sem))
                copies.append(pltpu.async_copy(tab.at[idx_r.at[ic]], rbuf.at[dst], sem))
                copies.append(pltpu.async_copy(tab.at[idx_t.at[ic]], tbuf.at[dst], sem))
            for cp in copies:
                cp.wait()

            def group(g, carry):
                rows = g * 16 + riota

                def distk(k, acc):
                    kk = jnp.full((16,), k, jnp.int32)
                    hv = plsc.load_gather(hbuf, [rows, kk])
                    rv = plsc.load_gather(rbuf, [rows, kk + _DIM])
                    tv = plsc.load_gather(tbuf, [rows, kk])
                    return acc + jnp.abs(hv + rv - tv)

                acc = lax.fori_loop(0, _DIM, distk, zeros)
                dist_buf[pl.ds(sub * _SB + g * 16, 16)] = acc
                return carry

            lax.fori_loop(0, _NG, group, 0)

    compute_set(pos_h, pos_r, pos_t, pd_buf)
    compute_set(neg_h, neg_r, neg_t, nd_buf)

    def loss_group(g, carry):
        sl = pl.ds(g * 16, 16)
        loss_buf[sl] = jnp.maximum(pd_buf[sl] - nd_buf[sl] + _MARGIN, 0.0)
        return carry

    lax.fori_loop(0, _BW // 16, loss_group, 0)

    obase = pl.ds(wid * _BW, _BW)
    pltpu.sync_copy(loss_buf, out_loss.at[obase])
    pltpu.sync_copy(pd_buf, out_pd.at[obase])
    pltpu.sync_copy(nd_buf, out_nd.at[obase])


_sc_call = pl.kernel(
    _sc_body,
    out_type=(
        jax.ShapeDtypeStruct((_BATCH,), jnp.float32),
        jax.ShapeDtypeStruct((_BATCH,), jnp.float32),
        jax.ShapeDtypeStruct((_BATCH,), jnp.float32),
    ),
    mesh=plsc.VectorSubcoreMesh(
        core_axis_name="c", subcore_axis_name="s",
        num_cores=_NC, num_subcores=_NS),
    scratch_types=[
        pltpu.VMEM((_NCH, _CH), jnp.int32),
        pltpu.VMEM((_NCH, _CH), jnp.int32),
        pltpu.VMEM((_NCH, _CH), jnp.int32),
        pltpu.VMEM((_SB, _PITCH), jnp.float32),
        pltpu.VMEM((_SB, _PITCH), jnp.float32),
        pltpu.VMEM((_SB, _PITCH), jnp.float32),
        pltpu.VMEM((_BW,), jnp.float32),
        pltpu.VMEM((_BW,), jnp.float32),
        pltpu.VMEM((_BW,), jnp.float32),
        pltpu.SemaphoreType.DMA,
    ],
    compiler_params=pltpu.CompilerParams(
        needs_layout_passes=False, use_tc_tiling_on_sc=False),
)


@jax.jit
def kernel(positive_triplets, negative_triplets, entities_emb, relations_emb):
    tab = _tc_stage(entities_emb.T, relations_emb.T)
    pos = positive_triplets.astype(jnp.int32)
    neg = negative_triplets.astype(jnp.int32)
    cols = lambda t, j: t[:, j].reshape(_NW * _NCH, _CH)
    return _sc_call(
        cols(pos, 0), cols(pos, 1), cols(pos, 2),
        cols(neg, 0), cols(neg, 1), cols(neg, 2),
        tab)


# speedup vs baseline: 1.6059x; 1.3117x over previous
import functools

import jax
import jax.numpy as jnp
from jax import lax
from jax.experimental import pallas as pl
from jax.experimental.pallas import tpu as pltpu
from jax.experimental.pallas import tpu_sc as plsc

_N_ENT = 1000001
_DIM = 64
_PITCH = 128
_MARGIN = 1.0
_BATCH = 16384

_NC = 2
_NS = 16
_NW = _NC * _NS
_BW = _BATCH // _NW
_SB = 256
_CH = 128
_NCH = _BW // _CH
_NG = _SB // 16

_TBLK = 2048
_TGRID = (_N_ENT + _TBLK - 1) // _TBLK


def _tc_stage_body(ent_ref, rel_ref, o_ref):
    eye = jnp.eye(_DIM, dtype=jnp.float32)
    tr = lambda m: lax.dot_general(m, eye, (((0,), (0,)), ((), ())),
                                   preferred_element_type=jnp.float32)
    x = ent_ref[...]
    inv = lax.rsqrt(jnp.sum(x * x, axis=0, keepdims=True))
    o_ref[...] = jnp.concatenate(
        [tr(x * inv), tr(rel_ref[...])], axis=1)


_tc_stage = pl.pallas_call(
    _tc_stage_body,
    grid=(_TGRID,),
    in_specs=[
        pl.BlockSpec((_DIM, _TBLK), lambda i: (0, i)),
        pl.BlockSpec((_DIM, _TBLK), lambda i: (0, i)),
    ],
    out_specs=pl.BlockSpec((_TBLK, _PITCH), lambda i: (i, 0)),
    out_shape=jax.ShapeDtypeStruct((_N_ENT, _PITCH), jnp.float32),
)


def _sc_body(pos_h, pos_r, pos_t, neg_h, neg_r, neg_t, tab,
             out_loss, out_pd, out_nd,
             idx_h, idx_r, idx_t, hbuf, rbuf, tbuf, pd_buf, nd_buf, loss_buf,
             sem):
    c = lax.axis_index("c")
    s = lax.axis_index("s")
    wid = s * _NC + c
    riota = lax.iota(jnp.int32, 16)
    zeros = jnp.zeros((16,), jnp.float32)
    half0 = jnp.zeros((16,), jnp.int32)
    half1 = jnp.ones((16,), jnp.int32)

    def compute_set(ih_hbm, ir_hbm, it_hbm, dist_buf):
        base = wid * _NCH
        pltpu.sync_copy(ih_hbm.at[pl.ds(base, _NCH)], idx_h)
        pltpu.sync_copy(ir_hbm.at[pl.ds(base, _NCH)], idx_r)
        pltpu.sync_copy(it_hbm.at[pl.ds(base, _NCH)], idx_t)
        for sub in range(_BW // _SB):
            copies = []
            for k in range(_SB // _CH):
                dst = pl.ds(k * _CH, _CH)
                ic = sub * (_SB // _CH) + k
                copies.append(pltpu.async_copy(tab.at[idx_h.at[ic]], hbuf.at[dst], sem))
                copies.append(pltpu.async_copy(tab.at[idx_r.at[ic]], rbuf.at[dst], sem))
                copies.append(pltpu.async_copy(tab.at[idx_t.at[ic]], tbuf.at[dst], sem))
            for cp in copies:
                cp.wait()

            def group(g, carry):
                rows = g * 16 + riota

                def distk(k, acc):
                    kk = jnp.full((16,), k, jnp.int32)
                    hv = plsc.load_gather(hbuf, [rows, kk])
                    rv = plsc.load_gather(rbuf, [rows, kk + _DIM])
                    tv = plsc.load_gather(tbuf, [rows, kk])
                    return acc + jnp.abs(hv + rv - tv)

                acc = lax.fori_loop(0, _DIM, distk, zeros)
                dist_buf[pl.ds(sub * _SB + g * 16, 16)] = acc
                return carry

            lax.fori_loop(0, _NG, group, 0)

    compute_set(pos_h, pos_r, pos_t, pd_buf)
    compute_set(neg_h, neg_r, neg_t, nd_buf)

    def loss_group(g, carry):
        sl = pl.ds(g * 16, 16)
        loss_buf[sl] = jnp.maximum(pd_buf[sl] - nd_buf[sl] + _MARGIN, 0.0)
        return carry

    lax.fori_loop(0, _BW // 16, loss_group, 0)

    obase = pl.ds(wid * _BW, _BW)
    pltpu.sync_copy(loss_buf, out_loss.at[obase])
    pltpu.sync_copy(pd_buf, out_pd.at[obase])
    pltpu.sync_copy(nd_buf, out_nd.at[obase])


_sc_call = pl.kernel(
    _sc_body,
    out_type=(
        jax.ShapeDtypeStruct((_BATCH,), jnp.float32),
        jax.ShapeDtypeStruct((_BATCH,), jnp.float32),
        jax.ShapeDtypeStruct((_BATCH,), jnp.float32),
    ),
    mesh=plsc.VectorSubcoreMesh(
        core_axis_name="c", subcore_axis_name="s",
        num_cores=_NC, num_subcores=_NS),
    scratch_types=[
        pltpu.VMEM((_NCH, _CH), jnp.int32),
        pltpu.VMEM((_NCH, _CH), jnp.int32),
        pltpu.VMEM((_NCH, _CH), jnp.int32),
        pltpu.VMEM((_SB, _PITCH), jnp.float32),
        pltpu.VMEM((_SB, _PITCH), jnp.float32),
        pltpu.VMEM((_SB, _PITCH), jnp.float32),
        pltpu.VMEM((_BW,), jnp.float32),
        pltpu.VMEM((_BW,), jnp.float32),
        pltpu.VMEM((_BW,), jnp.float32),
        pltpu.SemaphoreType.DMA,
    ],
    compiler_params=pltpu.CompilerParams(
        needs_layout_passes=False, use_tc_tiling_on_sc=False),
)


@jax.jit
def kernel(positive_triplets, negative_triplets, entities_emb, relations_emb):
    tab = _tc_stage(entities_emb.T, relations_emb.T)
    pos = positive_triplets.astype(jnp.int32)
    neg = negative_triplets.astype(jnp.int32)
    cols = lambda t, j: t[:, j].reshape(_NW * _NCH, _CH)
    return _sc_call(
        cols(pos, 0), cols(pos, 1), cols(pos, 2),
        cols(neg, 0), cols(neg, 1), cols(neg, 2),
        tab)


# speedup vs baseline: 1.9486x; 1.2134x over previous
import functools

import jax
import jax.numpy as jnp
from jax import lax
from jax.experimental import pallas as pl
from jax.experimental.pallas import tpu as pltpu
from jax.experimental.pallas import tpu_sc as plsc

_N_ENT = 1000001
_DIM = 64
_PITCH = 128
_MARGIN = 1.0
_BATCH = 16384

_NC = 2
_NS = 16
_NW = _NC * _NS
_BW = _BATCH // _NW
_SB = 256
_CH = 128
_NCH = _BW // _CH
_NG = _SB // 16

_TBLK = 4096
_TGRID = (_N_ENT + _TBLK - 1) // _TBLK


def _tc_stage_body(ent_ref, rel_ref, o_ref):
    eye = jnp.eye(_DIM, dtype=jnp.float32)
    tr = lambda m: lax.dot_general(m, eye, (((0,), (0,)), ((), ())),
                                   preferred_element_type=jnp.float32)
    x = ent_ref[...]
    inv = lax.rsqrt(jnp.sum(x * x, axis=0, keepdims=True))
    o_ref[...] = jnp.concatenate(
        [(x * inv).T, rel_ref[...].T], axis=1)


_tc_stage = pl.pallas_call(
    _tc_stage_body,
    grid=(_TGRID,),
    in_specs=[
        pl.BlockSpec((_DIM, _TBLK), lambda i: (0, i)),
        pl.BlockSpec((_DIM, _TBLK), lambda i: (0, i)),
    ],
    out_specs=pl.BlockSpec((_TBLK, _PITCH), lambda i: (i, 0)),
    out_shape=jax.ShapeDtypeStruct((_N_ENT, _PITCH), jnp.float32),
)


def _sc_body(pos_h, pos_r, pos_t, neg_h, neg_r, neg_t, tab,
             out_loss, out_pd, out_nd,
             idx_h, idx_r, idx_t, hbuf, rbuf, tbuf, pd_buf, nd_buf, loss_buf,
             sem):
    c = lax.axis_index("c")
    s = lax.axis_index("s")
    wid = s * _NC + c
    riota = lax.iota(jnp.int32, 16)
    zeros = jnp.zeros((16,), jnp.float32)
    half0 = jnp.zeros((16,), jnp.int32)
    half1 = jnp.ones((16,), jnp.int32)

    def compute_set(ih_hbm, ir_hbm, it_hbm, dist_buf):
        base = wid * _NCH
        pltpu.sync_copy(ih_hbm.at[pl.ds(base, _NCH)], idx_h)
        pltpu.sync_copy(ir_hbm.at[pl.ds(base, _NCH)], idx_r)
        pltpu.sync_copy(it_hbm.at[pl.ds(base, _NCH)], idx_t)
        for sub in range(_BW // _SB):
            copies = []
            for k in range(_SB // _CH):
                dst = pl.ds(k * _CH, _CH)
                ic = sub * (_SB // _CH) + k
                copies.append(pltpu.async_copy(tab.at[idx_h.at[ic]], hbuf.at[dst], sem))
                copies.append(pltpu.async_copy(tab.at[idx_r.at[ic]], rbuf.at[dst], sem))
                copies.append(pltpu.async_copy(tab.at[idx_t.at[ic]], tbuf.at[dst], sem))
            for cp in copies:
                cp.wait()

            def group(g, carry):
                rows = g * 16 + riota

                def distk(k, acc):
                    kk = jnp.full((16,), k, jnp.int32)
                    hv = plsc.load_gather(hbuf, [rows, kk])
                    rv = plsc.load_gather(rbuf, [rows, kk + _DIM])
                    tv = plsc.load_gather(tbuf, [rows, kk])
                    return acc + jnp.abs(hv + rv - tv)

                acc = lax.fori_loop(0, _DIM, distk, zeros)
                dist_buf[pl.ds(sub * _SB + g * 16, 16)] = acc
                return carry

            lax.fori_loop(0, _NG, group, 0)

    compute_set(pos_h, pos_r, pos_t, pd_buf)
    compute_set(neg_h, neg_r, neg_t, nd_buf)

    def loss_group(g, carry):
        sl = pl.ds(g * 16, 16)
        loss_buf[sl] = jnp.maximum(pd_buf[sl] - nd_buf[sl] + _MARGIN, 0.0)
        return carry

    lax.fori_loop(0, _BW // 16, loss_group, 0)

    obase = pl.ds(wid * _BW, _BW)
    pltpu.sync_copy(loss_buf, out_loss.at[obase])
    pltpu.sync_copy(pd_buf, out_pd.at[obase])
    pltpu.sync_copy(nd_buf, out_nd.at[obase])


_sc_call = pl.kernel(
    _sc_body,
    out_type=(
        jax.ShapeDtypeStruct((_BATCH,), jnp.float32),
        jax.ShapeDtypeStruct((_BATCH,), jnp.float32),
        jax.ShapeDtypeStruct((_BATCH,), jnp.float32),
    ),
    mesh=plsc.VectorSubcoreMesh(
        core_axis_name="c", subcore_axis_name="s",
        num_cores=_NC, num_subcores=_NS),
    scratch_types=[
        pltpu.VMEM((_NCH, _CH), jnp.int32),
        pltpu.VMEM((_NCH, _CH), jnp.int32),
        pltpu.VMEM((_NCH, _CH), jnp.int32),
        pltpu.VMEM((_SB, _PITCH), jnp.float32),
        pltpu.VMEM((_SB, _PITCH), jnp.float32),
        pltpu.VMEM((_SB, _PITCH), jnp.float32),
        pltpu.VMEM((_BW,), jnp.float32),
        pltpu.VMEM((_BW,), jnp.float32),
        pltpu.VMEM((_BW,), jnp.float32),
        pltpu.SemaphoreType.DMA,
    ],
    compiler_params=pltpu.CompilerParams(
        needs_layout_passes=False, use_tc_tiling_on_sc=False),
)


@jax.jit
def kernel(positive_triplets, negative_triplets, entities_emb, relations_emb):
    tab = _tc_stage(entities_emb.T, relations_emb.T)
    pos = positive_triplets.astype(jnp.int32)
    neg = negative_triplets.astype(jnp.int32)
    cols = lambda t, j: t[:, j].reshape(_NW * _NCH, _CH)
    return _sc_call(
        cols(pos, 0), cols(pos, 1), cols(pos, 2),
        cols(neg, 0), cols(neg, 1), cols(neg, 2),
        tab)


# speedup vs baseline: 2.1830x; 1.1203x over previous
import functools

import jax
import jax.numpy as jnp
from jax import lax
from jax.experimental import pallas as pl
from jax.experimental.pallas import tpu as pltpu
from jax.experimental.pallas import tpu_sc as plsc

_N_ENT = 1000001
_DIM = 64
_PITCH = 128
_MARGIN = 1.0
_BATCH = 16384

_NC = 2
_NS = 16
_NW = _NC * _NS
_BW = _BATCH // _NW
_SB = 256
_CH = 128
_NCH = _BW // _CH
_NG = _SB // 16

_TBLK = 8192
_TGRID = (_N_ENT + _TBLK - 1) // _TBLK


def _tc_stage_body(ent_ref, rel_ref, o_ref):
    eye = jnp.eye(_DIM, dtype=jnp.float32)
    tr = lambda m: lax.dot_general(m, eye, (((0,), (0,)), ((), ())),
                                   preferred_element_type=jnp.float32)
    x = ent_ref[...]
    inv = lax.rsqrt(jnp.sum(x * x, axis=0, keepdims=True))
    o_ref[...] = jnp.concatenate(
        [(x * inv).T, rel_ref[...].T], axis=1)


_tc_stage = pl.pallas_call(
    _tc_stage_body,
    grid=(_TGRID,),
    in_specs=[
        pl.BlockSpec((_DIM, _TBLK), lambda i: (0, i)),
        pl.BlockSpec((_DIM, _TBLK), lambda i: (0, i)),
    ],
    out_specs=pl.BlockSpec((_TBLK, _PITCH), lambda i: (i, 0)),
    out_shape=jax.ShapeDtypeStruct((_N_ENT, _PITCH), jnp.float32),
)


def _sc_body(pos_h, pos_r, pos_t, neg_h, neg_r, neg_t, tab,
             out_loss, out_pd, out_nd,
             idx_h, idx_r, idx_t, hbuf, rbuf, tbuf, pd_buf, nd_buf, loss_buf,
             sem):
    c = lax.axis_index("c")
    s = lax.axis_index("s")
    wid = s * _NC + c
    riota = lax.iota(jnp.int32, 16)
    zeros = jnp.zeros((16,), jnp.float32)
    half0 = jnp.zeros((16,), jnp.int32)
    half1 = jnp.ones((16,), jnp.int32)

    def compute_set(ih_hbm, ir_hbm, it_hbm, dist_buf):
        base = wid * _NCH
        pltpu.sync_copy(ih_hbm.at[pl.ds(base, _NCH)], idx_h)
        pltpu.sync_copy(ir_hbm.at[pl.ds(base, _NCH)], idx_r)
        pltpu.sync_copy(it_hbm.at[pl.ds(base, _NCH)], idx_t)
        for sub in range(_BW // _SB):
            copies = []
            for k in range(_SB // _CH):
                dst = pl.ds(k * _CH, _CH)
                ic = sub * (_SB // _CH) + k
                copies.append(pltpu.async_copy(tab.at[idx_h.at[ic]], hbuf.at[dst], sem))
                copies.append(pltpu.async_copy(tab.at[idx_r.at[ic]], rbuf.at[dst], sem))
                copies.append(pltpu.async_copy(tab.at[idx_t.at[ic]], tbuf.at[dst], sem))
            for cp in copies:
                cp.wait()

            def group(g, carry):
                rows = g * 16 + riota

                def distk(k, acc):
                    kk = jnp.full((16,), k, jnp.int32)
                    hv = plsc.load_gather(hbuf, [rows, kk])
                    rv = plsc.load_gather(rbuf, [rows, kk + _DIM])
                    tv = plsc.load_gather(tbuf, [rows, kk])
                    return acc + jnp.abs(hv + rv - tv)

                acc = lax.fori_loop(0, _DIM, distk, zeros)
                dist_buf[pl.ds(sub * _SB + g * 16, 16)] = acc
                return carry

            lax.fori_loop(0, _NG, group, 0)

    compute_set(pos_h, pos_r, pos_t, pd_buf)
    compute_set(neg_h, neg_r, neg_t, nd_buf)

    def loss_group(g, carry):
        sl = pl.ds(g * 16, 16)
        loss_buf[sl] = jnp.maximum(pd_buf[sl] - nd_buf[sl] + _MARGIN, 0.0)
        return carry

    lax.fori_loop(0, _BW // 16, loss_group, 0)

    obase = pl.ds(wid * _BW, _BW)
    pltpu.sync_copy(loss_buf, out_loss.at[obase])
    pltpu.sync_copy(pd_buf, out_pd.at[obase])
    pltpu.sync_copy(nd_buf, out_nd.at[obase])


_sc_call = pl.kernel(
    _sc_body,
    out_type=(
        jax.ShapeDtypeStruct((_BATCH,), jnp.float32),
        jax.ShapeDtypeStruct((_BATCH,), jnp.float32),
        jax.ShapeDtypeStruct((_BATCH,), jnp.float32),
    ),
    mesh=plsc.VectorSubcoreMesh(
        core_axis_name="c", subcore_axis_name="s",
        num_cores=_NC, num_subcores=_NS),
    scratch_types=[
        pltpu.VMEM((_NCH, _CH), jnp.int32),
        pltpu.VMEM((_NCH, _CH), jnp.int32),
        pltpu.VMEM((_NCH, _CH), jnp.int32),
        pltpu.VMEM((_SB, _PITCH), jnp.float32),
        pltpu.VMEM((_SB, _PITCH), jnp.float32),
        pltpu.VMEM((_SB, _PITCH), jnp.float32),
        pltpu.VMEM((_BW,), jnp.float32),
        pltpu.VMEM((_BW,), jnp.float32),
        pltpu.VMEM((_BW,), jnp.float32),
        pltpu.SemaphoreType.DMA,
    ],
    compiler_params=pltpu.CompilerParams(
        needs_layout_passes=False, use_tc_tiling_on_sc=False),
)


@jax.jit
def kernel(positive_triplets, negative_triplets, entities_emb, relations_emb):
    tab = _tc_stage(entities_emb.T, relations_emb.T)
    pos = positive_triplets.astype(jnp.int32)
    neg = negative_triplets.astype(jnp.int32)
    cols = lambda t, j: t[:, j].reshape(_NW * _NCH, _CH)
    return _sc_call(
        cols(pos, 0), cols(pos, 1), cols(pos, 2),
        cols(neg, 0), cols(neg, 1), cols(neg, 2),
        tab)


# speedup vs baseline: 2.2994x; 1.0533x over previous
import functools

import jax
import jax.numpy as jnp
from jax import lax
from jax.experimental import pallas as pl
from jax.experimental.pallas import tpu as pltpu
from jax.experimental.pallas import tpu_sc as plsc

_N_ENT = 1000001
_DIM = 64
_PITCH = 128
_MARGIN = 1.0
_BATCH = 16384

_NC = 2
_NS = 16
_NW = _NC * _NS
_BW = _BATCH // _NW
_SB = 256
_CH = 128
_NCH = _BW // _CH
_NG = _SB // 16

_TBLK = 16384
_TGRID = (_N_ENT + _TBLK - 1) // _TBLK


def _tc_stage_body(ent_ref, rel_ref, o_ref):
    eye = jnp.eye(_DIM, dtype=jnp.float32)
    tr = lambda m: lax.dot_general(m, eye, (((0,), (0,)), ((), ())),
                                   preferred_element_type=jnp.float32)
    x = ent_ref[...]
    inv = lax.rsqrt(jnp.sum(x * x, axis=0, keepdims=True))
    o_ref[...] = jnp.concatenate(
        [(x * inv).T, rel_ref[...].T], axis=1)


_tc_stage = pl.pallas_call(
    _tc_stage_body,
    grid=(_TGRID,),
    in_specs=[
        pl.BlockSpec((_DIM, _TBLK), lambda i: (0, i)),
        pl.BlockSpec((_DIM, _TBLK), lambda i: (0, i)),
    ],
    out_specs=pl.BlockSpec((_TBLK, _PITCH), lambda i: (i, 0)),
    out_shape=jax.ShapeDtypeStruct((_N_ENT, _PITCH), jnp.float32),
)


def _sc_body(pos_h, pos_r, pos_t, neg_h, neg_r, neg_t, tab,
             out_loss, out_pd, out_nd,
             idx_h, idx_r, idx_t, hbuf, rbuf, tbuf, pd_buf, nd_buf, loss_buf,
             sem):
    c = lax.axis_index("c")
    s = lax.axis_index("s")
    wid = s * _NC + c
    riota = lax.iota(jnp.int32, 16)
    zeros = jnp.zeros((16,), jnp.float32)
    half0 = jnp.zeros((16,), jnp.int32)
    half1 = jnp.ones((16,), jnp.int32)

    def compute_set(ih_hbm, ir_hbm, it_hbm, dist_buf):
        base = wid * _NCH
        pltpu.sync_copy(ih_hbm.at[pl.ds(base, _NCH)], idx_h)
        pltpu.sync_copy(ir_hbm.at[pl.ds(base, _NCH)], idx_r)
        pltpu.sync_copy(it_hbm.at[pl.ds(base, _NCH)], idx_t)
        for sub in range(_BW // _SB):
            copies = []
            for k in range(_SB // _CH):
                dst = pl.ds(k * _CH, _CH)
                ic = sub * (_SB // _CH) + k
                copies.append(pltpu.async_copy(tab.at[idx_h.at[ic]], hbuf.at[dst], sem))
                copies.append(pltpu.async_copy(tab.at[idx_r.at[ic]], rbuf.at[dst], sem))
                copies.append(pltpu.async_copy(tab.at[idx_t.at[ic]], tbuf.at[dst], sem))
            for cp in copies:
                cp.wait()

            def group(g, carry):
                rows = g * 16 + riota

                def distk(k, acc):
                    kk = jnp.full((16,), k, jnp.int32)
                    hv = plsc.load_gather(hbuf, [rows, kk])
                    rv = plsc.load_gather(rbuf, [rows, kk + _DIM])
                    tv = plsc.load_gather(tbuf, [rows, kk])
                    return acc + jnp.abs(hv + rv - tv)

                acc = lax.fori_loop(0, _DIM, distk, zeros)
                dist_buf[pl.ds(sub * _SB + g * 16, 16)] = acc
                return carry

            lax.fori_loop(0, _NG, group, 0)

    compute_set(pos_h, pos_r, pos_t, pd_buf)
    compute_set(neg_h, neg_r, neg_t, nd_buf)

    def loss_group(g, carry):
        sl = pl.ds(g * 16, 16)
        loss_buf[sl] = jnp.maximum(pd_buf[sl] - nd_buf[sl] + _MARGIN, 0.0)
        return carry

    lax.fori_loop(0, _BW // 16, loss_group, 0)

    obase = pl.ds(wid * _BW, _BW)
    pltpu.sync_copy(loss_buf, out_loss.at[obase])
    pltpu.sync_copy(pd_buf, out_pd.at[obase])
    pltpu.sync_copy(nd_buf, out_nd.at[obase])


_sc_call = pl.kernel(
    _sc_body,
    out_type=(
        jax.ShapeDtypeStruct((_BATCH,), jnp.float32),
        jax.ShapeDtypeStruct((_BATCH,), jnp.float32),
        jax.ShapeDtypeStruct((_BATCH,), jnp.float32),
    ),
    mesh=plsc.VectorSubcoreMesh(
        core_axis_name="c", subcore_axis_name="s",
        num_cores=_NC, num_subcores=_NS),
    scratch_types=[
        pltpu.VMEM((_NCH, _CH), jnp.int32),
        pltpu.VMEM((_NCH, _CH), jnp.int32),
        pltpu.VMEM((_NCH, _CH), jnp.int32),
        pltpu.VMEM((_SB, _PITCH), jnp.float32),
        pltpu.VMEM((_SB, _PITCH), jnp.float32),
        pltpu.VMEM((_SB, _PITCH), jnp.float32),
        pltpu.VMEM((_BW,), jnp.float32),
        pltpu.VMEM((_BW,), jnp.float32),
        pltpu.VMEM((_BW,), jnp.float32),
        pltpu.SemaphoreType.DMA,
    ],
    compiler_params=pltpu.CompilerParams(
        needs_layout_passes=False, use_tc_tiling_on_sc=False),
)


@jax.jit
def kernel(positive_triplets, negative_triplets, entities_emb, relations_emb):
    tab = _tc_stage(entities_emb.T, relations_emb.T)
    pos = positive_triplets.astype(jnp.int32)
    neg = negative_triplets.astype(jnp.int32)
    cols = lambda t, j: t[:, j].reshape(_NW * _NCH, _CH)
    return _sc_call(
        cols(pos, 0), cols(pos, 1), cols(pos, 2),
        cols(neg, 0), cols(neg, 1), cols(neg, 2),
        tab)


# speedup vs baseline: 2.7539x; 1.1977x over previous
import functools

import jax
import jax.numpy as jnp
from jax import lax
from jax.experimental import pallas as pl
from jax.experimental.pallas import tpu as pltpu
from jax.experimental.pallas import tpu_sc as plsc

_N_ENT = 1000001
_DIM = 64
_PITCH = 128
_MARGIN = 1.0
_BATCH = 16384

_NC = 2
_NS = 16
_NW = _NC * _NS
_BW = _BATCH // _NW
_SB = 256
_CH = 128
_NCH = _BW // _CH
_NG = _SB // 16

_TBLK = 16384
_TGRID = (_N_ENT + _TBLK - 1) // _TBLK


def _tc_stage_body(ent_ref, rel_ref, o_ref):
    eye = jnp.eye(_DIM, dtype=jnp.float32)
    tr = lambda m: lax.dot_general(m, eye, (((0,), (0,)), ((), ())),
                                   preferred_element_type=jnp.float32)
    x = ent_ref[...]
    inv = lax.rsqrt(jnp.sum(x * x, axis=0, keepdims=True))
    o_ref[...] = jnp.concatenate(
        [x * inv, rel_ref[...]], axis=0).T


_tc_stage = pl.pallas_call(
    _tc_stage_body,
    grid=(_TGRID,),
    in_specs=[
        pl.BlockSpec((_DIM, _TBLK), lambda i: (0, i)),
        pl.BlockSpec((_DIM, _TBLK), lambda i: (0, i)),
    ],
    out_specs=pl.BlockSpec((_TBLK, _PITCH), lambda i: (i, 0)),
    out_shape=jax.ShapeDtypeStruct((_N_ENT, _PITCH), jnp.float32),
)


def _sc_body(pos_h, pos_r, pos_t, neg_h, neg_r, neg_t, tab,
             out_loss, out_pd, out_nd,
             idx_h, idx_r, idx_t, hbuf, rbuf, tbuf, pd_buf, nd_buf, loss_buf,
             sem):
    c = lax.axis_index("c")
    s = lax.axis_index("s")
    wid = s * _NC + c
    riota = lax.iota(jnp.int32, 16)
    zeros = jnp.zeros((16,), jnp.float32)
    half0 = jnp.zeros((16,), jnp.int32)
    half1 = jnp.ones((16,), jnp.int32)

    def compute_set(ih_hbm, ir_hbm, it_hbm, dist_buf):
        base = wid * _NCH
        pltpu.sync_copy(ih_hbm.at[pl.ds(base, _NCH)], idx_h)
        pltpu.sync_copy(ir_hbm.at[pl.ds(base, _NCH)], idx_r)
        pltpu.sync_copy(it_hbm.at[pl.ds(base, _NCH)], idx_t)
        for sub in range(_BW // _SB):
            copies = []
            for k in range(_SB // _CH):
                dst = pl.ds(k * _CH, _CH)
                ic = sub * (_SB // _CH) + k
                copies.append(pltpu.async_copy(tab.at[idx_h.at[ic]], hbuf.at[dst], sem))
                copies.append(pltpu.async_copy(tab.at[idx_r.at[ic]], rbuf.at[dst], sem))
                copies.append(pltpu.async_copy(tab.at[idx_t.at[ic]], tbuf.at[dst], sem))
            for cp in copies:
                cp.wait()

            def group(g, carry):
                rows = g * 16 + riota

                def distk(k, acc):
                    kk = jnp.full((16,), k, jnp.int32)
                    hv = plsc.load_gather(hbuf, [rows, kk])
                    rv = plsc.load_gather(rbuf, [rows, kk + _DIM])
                    tv = plsc.load_gather(tbuf, [rows, kk])
                    return acc + jnp.abs(hv + rv - tv)

                acc = lax.fori_loop(0, _DIM, distk, zeros)
                dist_buf[pl.ds(sub * _SB + g * 16, 16)] = acc
                return carry

            lax.fori_loop(0, _NG, group, 0)

    compute_set(pos_h, pos_r, pos_t, pd_buf)
    compute_set(neg_h, neg_r, neg_t, nd_buf)

    def loss_group(g, carry):
        sl = pl.ds(g * 16, 16)
        loss_buf[sl] = jnp.maximum(pd_buf[sl] - nd_buf[sl] + _MARGIN, 0.0)
        return carry

    lax.fori_loop(0, _BW // 16, loss_group, 0)

    obase = pl.ds(wid * _BW, _BW)
    pltpu.sync_copy(loss_buf, out_loss.at[obase])
    pltpu.sync_copy(pd_buf, out_pd.at[obase])
    pltpu.sync_copy(nd_buf, out_nd.at[obase])


_sc_call = pl.kernel(
    _sc_body,
    out_type=(
        jax.ShapeDtypeStruct((_BATCH,), jnp.float32),
        jax.ShapeDtypeStruct((_BATCH,), jnp.float32),
        jax.ShapeDtypeStruct((_BATCH,), jnp.float32),
    ),
    mesh=plsc.VectorSubcoreMesh(
        core_axis_name="c", subcore_axis_name="s",
        num_cores=_NC, num_subcores=_NS),
    scratch_types=[
        pltpu.VMEM((_NCH, _CH), jnp.int32),
        pltpu.VMEM((_NCH, _CH), jnp.int32),
        pltpu.VMEM((_NCH, _CH), jnp.int32),
        pltpu.VMEM((_SB, _PITCH), jnp.float32),
        pltpu.VMEM((_SB, _PITCH), jnp.float32),
        pltpu.VMEM((_SB, _PITCH), jnp.float32),
        pltpu.VMEM((_BW,), jnp.float32),
        pltpu.VMEM((_BW,), jnp.float32),
        pltpu.VMEM((_BW,), jnp.float32),
        pltpu.SemaphoreType.DMA,
    ],
    compiler_params=pltpu.CompilerParams(
        needs_layout_passes=False, use_tc_tiling_on_sc=False),
)


@jax.jit
def kernel(positive_triplets, negative_triplets, entities_emb, relations_emb):
    tab = _tc_stage(entities_emb.T, relations_emb.T)
    pos = positive_triplets.astype(jnp.int32)
    neg = negative_triplets.astype(jnp.int32)
    cols = lambda t, j: t[:, j].reshape(_NW * _NCH, _CH)
    return _sc_call(
        cols(pos, 0), cols(pos, 1), cols(pos, 2),
        cols(neg, 0), cols(neg, 1), cols(neg, 2),
        tab)


# speedup vs baseline: 2.8065x; 1.0191x over previous
import functools

import jax
import jax.numpy as jnp
from jax import lax
from jax.experimental import pallas as pl
from jax.experimental.pallas import tpu as pltpu
from jax.experimental.pallas import tpu_sc as plsc

_N_ENT = 1000001
_DIM = 64
_PITCH = 128
_MARGIN = 1.0
_BATCH = 16384

_NC = 2
_NS = 16
_NW = _NC * _NS
_BW = _BATCH // _NW
_SB = 256
_CH = 128
_NCH = _BW // _CH
_NG = _SB // 16

_TBLK = 16384
_TGRID = (_N_ENT + _TBLK - 1) // _TBLK


def _tc_stage_body(ent_ref, rel_ref, o_ref):
    x = ent_ref[...]
    inv = lax.rsqrt(jnp.sum(x * x, axis=0, keepdims=True))
    o_ref[...] = jnp.concatenate(
        [x * inv, rel_ref[...]], axis=0).T


_tc_stage = pl.pallas_call(
    _tc_stage_body,
    grid=(_TGRID,),
    in_specs=[
        pl.BlockSpec((_DIM, _TBLK), lambda i: (0, i)),
        pl.BlockSpec((_DIM, _TBLK), lambda i: (0, i)),
    ],
    out_specs=pl.BlockSpec((_TBLK, _PITCH), lambda i: (i, 0)),
    out_shape=jax.ShapeDtypeStruct((_N_ENT, _PITCH), jnp.float32),
)


def _sc_body(pos_h, pos_r, pos_t, neg_h, neg_r, neg_t, tab,
             out_loss, out_pd, out_nd,
             idx_h, idx_r, idx_t, hbuf, rbuf, tbuf, pd_buf, nd_buf, loss_buf,
             sem):
    c = lax.axis_index("c")
    s = lax.axis_index("s")
    wid = s * _NC + c
    riota = lax.iota(jnp.int32, 16)
    zeros = jnp.zeros((16,), jnp.float32)

    def compute_set(ih_hbm, ir_hbm, it_hbm, dist_buf):
        base = wid * _NCH
        pltpu.sync_copy(ih_hbm.at[pl.ds(base, _NCH)], idx_h)
        pltpu.sync_copy(ir_hbm.at[pl.ds(base, _NCH)], idx_r)
        pltpu.sync_copy(it_hbm.at[pl.ds(base, _NCH)], idx_t)
        for sub in range(_BW // _SB):
            copies = []
            for k in range(_SB // _CH):
                dst = pl.ds(k * _CH, _CH)
                ic = sub * (_SB // _CH) + k
                copies.append(pltpu.async_copy(tab.at[idx_h.at[ic]], hbuf.at[dst], sem))
                copies.append(pltpu.async_copy(tab.at[idx_r.at[ic]], rbuf.at[dst], sem))
                copies.append(pltpu.async_copy(tab.at[idx_t.at[ic]], tbuf.at[dst], sem))
            for cp in copies:
                cp.wait()

            def group(g, carry):
                rows = g * 16 + riota

                def distk(j, acc):
                    for dk in range(4):
                        kk = jnp.full((16,), j * 4 + dk, jnp.int32)
                        hv = plsc.load_gather(hbuf, [rows, kk])
                        rv = plsc.load_gather(rbuf, [rows, kk + _DIM])
                        tv = plsc.load_gather(tbuf, [rows, kk])
                        acc = acc + jnp.abs(hv + rv - tv)
                    return acc

                acc = lax.fori_loop(0, _DIM // 4, distk, zeros)
                dist_buf[pl.ds(sub * _SB + g * 16, 16)] = acc
                return carry

            lax.fori_loop(0, _NG, group, 0)

    compute_set(pos_h, pos_r, pos_t, pd_buf)
    compute_set(neg_h, neg_r, neg_t, nd_buf)

    def loss_group(g, carry):
        sl = pl.ds(g * 16, 16)
        loss_buf[sl] = jnp.maximum(pd_buf[sl] - nd_buf[sl] + _MARGIN, 0.0)
        return carry

    lax.fori_loop(0, _BW // 16, loss_group, 0)

    obase = pl.ds(wid * _BW, _BW)
    pltpu.sync_copy(loss_buf, out_loss.at[obase])
    pltpu.sync_copy(pd_buf, out_pd.at[obase])
    pltpu.sync_copy(nd_buf, out_nd.at[obase])


_sc_call = pl.kernel(
    _sc_body,
    out_type=(
        jax.ShapeDtypeStruct((_BATCH,), jnp.float32),
        jax.ShapeDtypeStruct((_BATCH,), jnp.float32),
        jax.ShapeDtypeStruct((_BATCH,), jnp.float32),
    ),
    mesh=plsc.VectorSubcoreMesh(
        core_axis_name="c", subcore_axis_name="s",
        num_cores=_NC, num_subcores=_NS),
    scratch_types=[
        pltpu.VMEM((_NCH, _CH), jnp.int32),
        pltpu.VMEM((_NCH, _CH), jnp.int32),
        pltpu.VMEM((_NCH, _CH), jnp.int32),
        pltpu.VMEM((_SB, _PITCH), jnp.float32),
        pltpu.VMEM((_SB, _PITCH), jnp.float32),
        pltpu.VMEM((_SB, _PITCH), jnp.float32),
        pltpu.VMEM((_BW,), jnp.float32),
        pltpu.VMEM((_BW,), jnp.float32),
        pltpu.VMEM((_BW,), jnp.float32),
        pltpu.SemaphoreType.DMA,
    ],
    compiler_params=pltpu.CompilerParams(
        needs_layout_passes=False, use_tc_tiling_on_sc=False),
)


@jax.jit
def kernel(positive_triplets, negative_triplets, entities_emb, relations_emb):
    tab = _tc_stage(entities_emb.T, relations_emb.T)
    pos = positive_triplets.astype(jnp.int32)
    neg = negative_triplets.astype(jnp.int32)
    cols = lambda t, j: t[:, j].reshape(_NW * _NCH, _CH)
    return _sc_call(
        cols(pos, 0), cols(pos, 1), cols(pos, 2),
        cols(neg, 0), cols(neg, 1), cols(neg, 2),
        tab)


# speedup vs baseline: 2.8944x; 1.0313x over previous
import functools

import jax
import jax.numpy as jnp
from jax import lax
from jax.experimental import pallas as pl
from jax.experimental.pallas import tpu as pltpu
from jax.experimental.pallas import tpu_sc as plsc

_N_ENT = 1000001
_DIM = 64
_PITCH = 128
_MARGIN = 1.0
_BATCH = 16384

_NC = 2
_NS = 16
_NW = _NC * _NS
_BW = _BATCH // _NW
_SB = 128
_NSB = _BW // _SB
_NCH = _NSB
_NG = _SB // 16

_TBLK = 16384
_TGRID = (_N_ENT + _TBLK - 1) // _TBLK


def _tc_stage_body(ent_ref, rel_ref, o_ref):
    x = ent_ref[...]
    inv = lax.rsqrt(jnp.sum(x * x, axis=0, keepdims=True))
    o_ref[...] = jnp.concatenate(
        [x * inv, rel_ref[...]], axis=0).T


_tc_stage = pl.pallas_call(
    _tc_stage_body,
    grid=(_TGRID,),
    in_specs=[
        pl.BlockSpec((_DIM, _TBLK), lambda i: (0, i)),
        pl.BlockSpec((_DIM, _TBLK), lambda i: (0, i)),
    ],
    out_specs=pl.BlockSpec((_TBLK, _PITCH), lambda i: (i, 0)),
    out_shape=jax.ShapeDtypeStruct((_N_ENT, _PITCH), jnp.float32),
)


def _sc_body(pos_h, pos_r, pos_t, neg_h, neg_r, neg_t, tab,
             out_loss, out_pd, out_nd,
             idx_h, idx_r, idx_t, hb0, rb0, tb0, hb1, rb1, tb1,
             pd_buf, nd_buf, loss_buf, sem):
    c = lax.axis_index("c")
    s = lax.axis_index("s")
    wid = s * _NC + c
    riota = lax.iota(jnp.int32, 16)
    zeros = jnp.zeros((16,), jnp.float32)
    bufs = ((hb0, rb0, tb0), (hb1, rb1, tb1))

    def compute_set(ih_hbm, ir_hbm, it_hbm, dist_buf):
        base = wid * _NCH
        pltpu.sync_copy(ih_hbm.at[pl.ds(base, _NCH)], idx_h)
        pltpu.sync_copy(ir_hbm.at[pl.ds(base, _NCH)], idx_r)
        pltpu.sync_copy(it_hbm.at[pl.ds(base, _NCH)], idx_t)

        def fire(sub):
            hb, rb, tb = bufs[sub % 2]
            dst = pl.ds(0, _SB)
            return (pltpu.async_copy(tab.at[idx_h.at[sub]], hb.at[dst], sem),
                    pltpu.async_copy(tab.at[idx_r.at[sub]], rb.at[dst], sem),
                    pltpu.async_copy(tab.at[idx_t.at[sub]], tb.at[dst], sem))

        pending = fire(0)
        for sub in range(_NSB):
            for cp in pending:
                cp.wait()
            if sub + 1 < _NSB:
                pending = fire(sub + 1)
            hb, rb, tb = bufs[sub % 2]

            def group(g, carry, hb=hb, rb=rb, tb=tb, sub=sub):
                rows = g * 16 + riota

                def distk(j, acc):
                    for dk in range(4):
                        kk = jnp.full((16,), j * 4 + dk, jnp.int32)
                        hv = plsc.load_gather(hb, [rows, kk])
                        rv = plsc.load_gather(rb, [rows, kk + _DIM])
                        tv = plsc.load_gather(tb, [rows, kk])
                        acc = acc + jnp.abs(hv + rv - tv)
                    return acc

                acc = lax.fori_loop(0, _DIM // 4, distk, zeros)
                dist_buf[pl.ds(sub * _SB + g * 16, 16)] = acc
                return carry

            lax.fori_loop(0, _NG, group, 0)

    compute_set(pos_h, pos_r, pos_t, pd_buf)
    compute_set(neg_h, neg_r, neg_t, nd_buf)

    def loss_group(g, carry):
        sl = pl.ds(g * 16, 16)
        loss_buf[sl] = jnp.maximum(pd_buf[sl] - nd_buf[sl] + _MARGIN, 0.0)
        return carry

    lax.fori_loop(0, _BW // 16, loss_group, 0)

    obase = pl.ds(wid * _BW, _BW)
    pltpu.sync_copy(loss_buf, out_loss.at[obase])
    pltpu.sync_copy(pd_buf, out_pd.at[obase])
    pltpu.sync_copy(nd_buf, out_nd.at[obase])


_sc_call = pl.kernel(
    _sc_body,
    out_type=(
        jax.ShapeDtypeStruct((_BATCH,), jnp.float32),
        jax.ShapeDtypeStruct((_BATCH,), jnp.float32),
        jax.ShapeDtypeStruct((_BATCH,), jnp.float32),
    ),
    mesh=plsc.VectorSubcoreMesh(
        core_axis_name="c", subcore_axis_name="s",
        num_cores=_NC, num_subcores=_NS),
    scratch_types=(
        [pltpu.VMEM((_NCH, _SB), jnp.int32)] * 3
        + [pltpu.VMEM((_SB, _PITCH), jnp.float32)] * 6
        + [pltpu.VMEM((_BW,), jnp.float32)] * 3
        + [pltpu.SemaphoreType.DMA]
    ),
    compiler_params=pltpu.CompilerParams(
        needs_layout_passes=False, use_tc_tiling_on_sc=False),
)


@jax.jit
def kernel(positive_triplets, negative_triplets, entities_emb, relations_emb):
    tab = _tc_stage(entities_emb.T, relations_emb.T)
    pos = positive_triplets.astype(jnp.int32)
    neg = negative_triplets.astype(jnp.int32)
    cols = lambda t, j: t[:, j].reshape(_NW * _NCH, _SB)
    return _sc_call(
        cols(pos, 0), cols(pos, 1), cols(pos, 2),
        cols(neg, 0), cols(neg, 1), cols(neg, 2),
        tab)


# speedup vs baseline: 2.9010x; 1.0023x over previous
import functools

import jax
import jax.numpy as jnp
from jax import lax
from jax.experimental import pallas as pl
from jax.experimental.pallas import tpu as pltpu
from jax.experimental.pallas import tpu_sc as plsc

_N_ENT = 1000001
_DIM = 64
_PITCH = 128
_MARGIN = 1.0
_BATCH = 16384

_NC = 2
_NS = 16
_NW = _NC * _NS
_BW = _BATCH // _NW
_SB = 128
_NSB = _BW // _SB
_NCH = _NSB
_NG = _SB // 16

_TBLK = 24576
_TGRID = (_N_ENT + _TBLK - 1) // _TBLK


def _tc_stage_body(ent_ref, rel_ref, o_ref):
    x = ent_ref[...]
    inv = lax.rsqrt(jnp.sum(x * x, axis=0, keepdims=True))
    o_ref[...] = jnp.concatenate(
        [x * inv, rel_ref[...]], axis=0).T


_tc_stage = pl.pallas_call(
    _tc_stage_body,
    grid=(_TGRID,),
    in_specs=[
        pl.BlockSpec((_DIM, _TBLK), lambda i: (0, i)),
        pl.BlockSpec((_DIM, _TBLK), lambda i: (0, i)),
    ],
    out_specs=pl.BlockSpec((_TBLK, _PITCH), lambda i: (i, 0)),
    out_shape=jax.ShapeDtypeStruct((_N_ENT, _PITCH), jnp.float32),
    compiler_params=pltpu.CompilerParams(
        vmem_limit_bytes=128 * 1024 * 1024),
)


def _sc_body(pos_h, pos_r, pos_t, neg_h, neg_r, neg_t, tab,
             out_loss, out_pd, out_nd,
             idx_h, idx_r, idx_t, hb0, rb0, tb0, hb1, rb1, tb1,
             pd_buf, nd_buf, loss_buf, sem):
    c = lax.axis_index("c")
    s = lax.axis_index("s")
    wid = s * _NC + c
    riota = lax.iota(jnp.int32, 16)
    zeros = jnp.zeros((16,), jnp.float32)
    bufs = ((hb0, rb0, tb0), (hb1, rb1, tb1))

    def compute_set(ih_hbm, ir_hbm, it_hbm, dist_buf):
        base = wid * _NCH
        pltpu.sync_copy(ih_hbm.at[pl.ds(base, _NCH)], idx_h)
        pltpu.sync_copy(ir_hbm.at[pl.ds(base, _NCH)], idx_r)
        pltpu.sync_copy(it_hbm.at[pl.ds(base, _NCH)], idx_t)

        def fire(sub):
            hb, rb, tb = bufs[sub % 2]
            dst = pl.ds(0, _SB)
            return (pltpu.async_copy(tab.at[idx_h.at[sub]], hb.at[dst], sem),
                    pltpu.async_copy(tab.at[idx_r.at[sub]], rb.at[dst], sem),
                    pltpu.async_copy(tab.at[idx_t.at[sub]], tb.at[dst], sem))

        pending = fire(0)
        for sub in range(_NSB):
            for cp in pending:
                cp.wait()
            if sub + 1 < _NSB:
                pending = fire(sub + 1)
            hb, rb, tb = bufs[sub % 2]

            def group(g, carry, hb=hb, rb=rb, tb=tb, sub=sub):
                rows = g * 16 + riota

                def distk(j, acc):
                    for dk in range(4):
                        kk = jnp.full((16,), j * 4 + dk, jnp.int32)
                        hv = plsc.load_gather(hb, [rows, kk])
                        rv = plsc.load_gather(rb, [rows, kk + _DIM])
                        tv = plsc.load_gather(tb, [rows, kk])
                        acc = acc + jnp.abs(hv + rv - tv)
                    return acc

                acc = lax.fori_loop(0, _DIM // 4, distk, zeros)
                dist_buf[pl.ds(sub * _SB + g * 16, 16)] = acc
                return carry

            lax.fori_loop(0, _NG, group, 0)

    compute_set(pos_h, pos_r, pos_t, pd_buf)
    compute_set(neg_h, neg_r, neg_t, nd_buf)

    def loss_group(g, carry):
        sl = pl.ds(g * 16, 16)
        loss_buf[sl] = jnp.maximum(pd_buf[sl] - nd_buf[sl] + _MARGIN, 0.0)
        return carry

    lax.fori_loop(0, _BW // 16, loss_group, 0)

    obase = pl.ds(wid * _BW, _BW)
    pltpu.sync_copy(loss_buf, out_loss.at[obase])
    pltpu.sync_copy(pd_buf, out_pd.at[obase])
    pltpu.sync_copy(nd_buf, out_nd.at[obase])


_sc_call = pl.kernel(
    _sc_body,
    out_type=(
        jax.ShapeDtypeStruct((_BATCH,), jnp.float32),
        jax.ShapeDtypeStruct((_BATCH,), jnp.float32),
        jax.ShapeDtypeStruct((_BATCH,), jnp.float32),
    ),
    mesh=plsc.VectorSubcoreMesh(
        core_axis_name="c", subcore_axis_name="s",
        num_cores=_NC, num_subcores=_NS),
    scratch_types=(
        [pltpu.VMEM((_NCH, _SB), jnp.int32)] * 3
        + [pltpu.VMEM((_SB, _PITCH), jnp.float32)] * 6
        + [pltpu.VMEM((_BW,), jnp.float32)] * 3
        + [pltpu.SemaphoreType.DMA]
    ),
    compiler_params=pltpu.CompilerParams(
        needs_layout_passes=False, use_tc_tiling_on_sc=False),
)


@jax.jit
def kernel(positive_triplets, negative_triplets, entities_emb, relations_emb):
    tab = _tc_stage(entities_emb.T, relations_emb.T)
    pos = positive_triplets.astype(jnp.int32)
    neg = negative_triplets.astype(jnp.int32)
    cols = lambda t, j: t[:, j].reshape(_NW * _NCH, _SB)
    return _sc_call(
        cols(pos, 0), cols(pos, 1), cols(pos, 2),
        cols(neg, 0), cols(neg, 1), cols(neg, 2),
        tab)


# speedup vs baseline: 2.9047x; 1.0013x over previous
import jax
import jax.numpy as jnp
from jax import lax
from jax.experimental import pallas as pl
from jax.experimental.pallas import tpu as pltpu
from jax.experimental.pallas import tpu_sc as plsc

_N_ENT = 1000001
_DIM = 64
_PITCH = 128
_MARGIN = 1.0
_BATCH = 16384

_NC = 2
_NS = 16
_NW = _NC * _NS
_BW = _BATCH // _NW
_SB = 128
_NSB = _BW // _SB
_NCH = _NSB
_NG = _SB // 16

_TBLK = 24576
_TGRID = (_N_ENT + _TBLK - 1) // _TBLK


def _tc_stage_body(ent_ref, rel_ref, o_ref):
    x = ent_ref[...]
    inv = lax.rsqrt(jnp.sum(x * x, axis=0, keepdims=True))
    o_ref[...] = jnp.concatenate(
        [x * inv, rel_ref[...]], axis=0).T


_tc_stage = pl.pallas_call(
    _tc_stage_body,
    grid=(_TGRID,),
    in_specs=[
        pl.BlockSpec((_DIM, _TBLK), lambda i: (0, i)),
        pl.BlockSpec((_DIM, _TBLK), lambda i: (0, i)),
    ],
    out_specs=pl.BlockSpec((_TBLK, _PITCH), lambda i: (i, 0)),
    out_shape=jax.ShapeDtypeStruct((_N_ENT, _PITCH), jnp.float32),
    compiler_params=pltpu.CompilerParams(
        vmem_limit_bytes=128 * 1024 * 1024),
)


def _sc_body(pos_h, pos_r, pos_t, neg_h, neg_r, neg_t, tab,
             out_loss, out_pd, out_nd,
             idx_h, idx_r, idx_t, hb0, rb0, tb0, hb1, rb1, tb1,
             pd_buf, nd_buf, loss_buf, sem):
    c = lax.axis_index("c")
    s = lax.axis_index("s")
    wid = s * _NC + c
    riota = lax.iota(jnp.int32, 16)
    zeros = jnp.zeros((16,), jnp.float32)
    bufs = ((hb0, rb0, tb0), (hb1, rb1, tb1))

    def compute_set(ih_hbm, ir_hbm, it_hbm, dist_buf):
        base = wid * _NCH
        pltpu.sync_copy(ih_hbm.at[pl.ds(base, _NCH)], idx_h)
        pltpu.sync_copy(ir_hbm.at[pl.ds(base, _NCH)], idx_r)
        pltpu.sync_copy(it_hbm.at[pl.ds(base, _NCH)], idx_t)

        def fire(sub):
            hb, rb, tb = bufs[sub % 2]
            dst = pl.ds(0, _SB)
            return (pltpu.async_copy(tab.at[idx_h.at[sub]], hb.at[dst], sem),
                    pltpu.async_copy(tab.at[idx_r.at[sub]], rb.at[dst], sem),
                    pltpu.async_copy(tab.at[idx_t.at[sub]], tb.at[dst], sem))

        pending = fire(0)
        for sub in range(_NSB):
            for cp in pending:
                cp.wait()
            if sub + 1 < _NSB:
                pending = fire(sub + 1)
            hb, rb, tb = bufs[sub % 2]

            def group(g, carry, hb=hb, rb=rb, tb=tb, sub=sub):
                rows = g * 16 + riota

                def distk(j, acc):
                    for dk in range(4):
                        kk = jnp.full((16,), j * 4 + dk, jnp.int32)
                        hv = plsc.load_gather(hb, [rows, kk])
                        rv = plsc.load_gather(rb, [rows, kk + _DIM])
                        tv = plsc.load_gather(tb, [rows, kk])
                        acc = acc + jnp.abs(hv + rv - tv)
                    return acc

                acc = lax.fori_loop(0, _DIM // 4, distk, zeros)
                dist_buf[pl.ds(sub * _SB + g * 16, 16)] = acc
                return carry

            lax.fori_loop(0, _NG, group, 0)

    compute_set(pos_h, pos_r, pos_t, pd_buf)
    compute_set(neg_h, neg_r, neg_t, nd_buf)

    def loss_group(g, carry):
        sl = pl.ds(g * 16, 16)
        loss_buf[sl] = jnp.maximum(pd_buf[sl] - nd_buf[sl] + _MARGIN, 0.0)
        return carry

    lax.fori_loop(0, _BW // 16, loss_group, 0)

    obase = pl.ds(wid * _BW, _BW)
    pltpu.sync_copy(loss_buf, out_loss.at[obase])
    pltpu.sync_copy(pd_buf, out_pd.at[obase])
    pltpu.sync_copy(nd_buf, out_nd.at[obase])


_sc_call = pl.kernel(
    _sc_body,
    out_type=(
        jax.ShapeDtypeStruct((_BATCH,), jnp.float32),
        jax.ShapeDtypeStruct((_BATCH,), jnp.float32),
        jax.ShapeDtypeStruct((_BATCH,), jnp.float32),
    ),
    mesh=plsc.VectorSubcoreMesh(
        core_axis_name="c", subcore_axis_name="s",
        num_cores=_NC, num_subcores=_NS),
    scratch_types=(
        [pltpu.VMEM((_NCH, _SB), jnp.int32)] * 3
        + [pltpu.VMEM((_SB, _PITCH), jnp.float32)] * 6
        + [pltpu.VMEM((_BW,), jnp.float32)] * 3
        + [pltpu.SemaphoreType.DMA]
    ),
    compiler_params=pltpu.CompilerParams(
        needs_layout_passes=False, use_tc_tiling_on_sc=False),
)


@jax.jit
def kernel(positive_triplets, negative_triplets, entities_emb, relations_emb):
    tab = _tc_stage(entities_emb.T, relations_emb.T)
    pos = positive_triplets.astype(jnp.int32)
    neg = negative_triplets.astype(jnp.int32)
    cols = lambda t, j: t[:, j].reshape(_NW * _NCH, _SB)
    return _sc_call(
        cols(pos, 0), cols(pos, 1), cols(pos, 2),
        cols(neg, 0), cols(neg, 1), cols(neg, 2),
        tab)
